# baseline (speedup 1.0000x reference)
import jax
import jax.numpy as jnp
from jax.experimental import pallas as pl

_B = 1024
_L = 64
_BB = 128


def _body(dec_ref, mov_ref, cat_ref, per_ref, com_ref,
          wd_ref, wm_ref, wc_ref, wp_ref, wco_ref, out_ref):
    def field(x_ref, wt_ref):
        xf = x_ref[...].astype(jnp.float32)
        y = jax.lax.dot_general(
            xf, wt_ref[...], (((1,), (0,)), ((), ())),
            preferred_element_type=jnp.float32)
        s = jnp.sum(xf, axis=1)
        return y, s

    yd, sd = field(dec_ref, wd_ref)
    ym, sm = field(mov_ref, wm_ref)
    yc, sc = field(cat_ref, wc_ref)
    yp, sp = field(per_ref, wp_ref)
    yco, sco = field(com_ref, wco_ref)

    def mean_div(y, s):
        nz = s != 0.0
        return jnp.where(nz[:, None], y / jnp.where(nz, s, 1.0)[:, None], y)

    yd = mean_div(yd, sd)
    yd = mean_div(yd, sm)
    yc = mean_div(yc, sc)
    yp = mean_div(yp, sp)
    yco = mean_div(yco, sco)

    out_ref[...] = jnp.concatenate((yd, ym, yc, yp, yco), axis=1)


def kernel(decade_idxs, movie_idxs, category_idxs, person_idxs, company_idxs,
           W_decade, W_movie, W_category, W_person, W_company):
    wts = [W.T for W in (W_decade, W_movie, W_category, W_person, W_company)]
    ks = [w.shape[0] for w in wts]
    grid = (_B // _BB,)
    in_specs = (
        [pl.BlockSpec((_BB, k), lambda i: (i, 0)) for k in ks]
        + [pl.BlockSpec((k, _L), lambda i: (0, 0)) for k in ks]
    )
    out = pl.pallas_call(
        _body,
        grid=grid,
        in_specs=in_specs,
        out_specs=pl.BlockSpec((_BB, 5 * _L), lambda i: (i, 0)),
        out_shape=jax.ShapeDtypeStruct((_B, 5 * _L), jnp.float32),
    )(decade_idxs, movie_idxs, category_idxs, person_idxs, company_idxs, *wts)
    return out
